# baseline (speedup 1.0000x reference)
import jax
import jax.numpy as jnp
from jax.experimental import pallas as pl

B = 16384
EMBED_DIM = 64
BLK = 512

def _tc_sub_body(x_ref, e_ref, o_ref):
    o_ref[...] = x_ref[...] - e_ref[...]

@jax.jit
def _batch_correct(x, batch_labels, batch_embed):
    eff = jnp.take(batch_embed, batch_labels, axis=0)
    return pl.pallas_call(
        _tc_sub_body,
        out_shape=jax.ShapeDtypeStruct((B, EMBED_DIM), jnp.float32),
        grid=(B // BLK,),
        in_specs=[
            pl.BlockSpec((BLK, EMBED_DIM), lambda i: (i, 0)),
            pl.BlockSpec((BLK, EMBED_DIM), lambda i: (i, 0)),
        ],
        out_specs=pl.BlockSpec((BLK, EMBED_DIM), lambda i: (i, 0)),
    )(x, eff)

def kernel(x, batch_labels, batch_embed):
    return _batch_correct(x, batch_labels.astype(jnp.int32), batch_embed)


# speedup vs baseline: 1.7210x; 1.7210x over previous
import jax
import jax.numpy as jnp
from jax import lax
from jax.experimental import pallas as pl
from jax.experimental.pallas import tpu as pltpu
from jax.experimental.pallas import tpu_sc as plsc

EMBED_DIM = 64
NUM_BATCHES = 1000
B = 16384

NC = 2
NS = 16
NW = NC * NS
B_PER_W = B // NW
N_SUB = 4
SUB = B_PER_W // N_SUB
PAIRS = SUB // 2
ROWS2 = B_PER_W // 2


def _sc_body(x_hbm, idx_hbm, table_hbm, out_hbm,
             idx_v, rows_v, x_v, x_sem, g_sems, o_sem):
    wid = lax.axis_index("s") * NC + lax.axis_index("c")
    base = wid * ROWS2

    x_copy = pltpu.async_copy(x_hbm.at[pl.ds(base, ROWS2)], x_v, x_sem)
    pltpu.sync_copy(idx_hbm.at[pl.ds(wid * B_PER_W, B_PER_W)], idx_v)
    gathers = [None, None]
    for j in range(2):
        gathers[j] = pltpu.async_copy(
            table_hbm.at[idx_v.at[pl.ds(j * SUB, SUB)]],
            rows_v.at[j], g_sems.at[j])
    x_copy.wait()

    stores = []
    for j in range(N_SUB):
        gathers[j % 2].wait()

        def sub_pair(q, _):
            r = j * PAIRS + q
            for c in range(EMBED_DIM // 16):
                sl = pl.ds(c * 16, 16)
                sh = pl.ds(EMBED_DIM + c * 16, 16)
                x_v[r, sl] = x_v[r, sl] - rows_v[j % 2, 2 * q, sl]
                x_v[r, sh] = x_v[r, sh] - rows_v[j % 2, 2 * q + 1, sl]
            return 0

        lax.fori_loop(0, PAIRS, sub_pair, 0)
        stores.append(pltpu.async_copy(
            x_v.at[pl.ds(j * PAIRS, PAIRS)],
            out_hbm.at[pl.ds(base + j * PAIRS, PAIRS)], o_sem))
        if j + 2 < N_SUB:
            gathers[j % 2] = pltpu.async_copy(
                table_hbm.at[idx_v.at[pl.ds((j + 2) * SUB, SUB)]],
                rows_v.at[j % 2], g_sems.at[j % 2])
    for s in stores:
        s.wait()


@jax.jit
def _batch_correct(x, batch_labels, batch_embed):
    mesh = plsc.VectorSubcoreMesh(core_axis_name="c", subcore_axis_name="s")
    tpad = jnp.pad(batch_embed, ((0, 0), (0, 128 - EMBED_DIM)))
    x2 = x.reshape(B // 2, 2 * EMBED_DIM)
    out2 = pl.kernel(
        _sc_body,
        out_type=jax.ShapeDtypeStruct((B // 2, 2 * EMBED_DIM), jnp.float32),
        mesh=mesh,
        scratch_types=[
            pltpu.VMEM((B_PER_W,), jnp.int32),
            pltpu.VMEM((2, SUB, 128), jnp.float32),
            pltpu.VMEM((ROWS2, 2 * EMBED_DIM), jnp.float32),
            pltpu.SemaphoreType.DMA,
            pltpu.SemaphoreType.DMA((2,)),
            pltpu.SemaphoreType.DMA,
        ],
        compiler_params=pltpu.CompilerParams(
            use_tc_tiling_on_sc=True,
            disable_bounds_checks=True,
            disable_semaphore_checks=True,
            skip_device_barrier=True,
        ),
    )(x2, batch_labels, tpad)
    return out2.reshape(B, EMBED_DIM)


def kernel(x, batch_labels, batch_embed):
    return _batch_correct(x, batch_labels.astype(jnp.int32), batch_embed)


# speedup vs baseline: 2.2735x; 1.3211x over previous
import jax
import jax.numpy as jnp
from jax import lax
from jax.experimental import pallas as pl
from jax.experimental.pallas import tpu as pltpu
from jax.experimental.pallas import tpu_sc as plsc

EMBED_DIM = 64
NUM_BATCHES = 1000
B = 16384

NC = 2
NS = 16
NW = NC * NS
B_PER_W = B // NW
N_SUB = 4
SUB = B_PER_W // N_SUB


def _sc_body(x_hbm, idx_hbm, table_hbm, out_hbm,
             idx_v, rows_v, x_v, x_sem, g_sems, o_sem):
    wid = lax.axis_index("s") * NC + lax.axis_index("c")
    base = wid * B_PER_W

    x_copy = pltpu.async_copy(x_hbm.at[pl.ds(base, B_PER_W)], x_v, x_sem)
    pltpu.sync_copy(idx_hbm.at[pl.ds(base, B_PER_W)], idx_v)
    gathers = [
        pltpu.async_copy(
            table_hbm.at[idx_v.at[pl.ds(j * SUB, SUB)]],
            rows_v.at[j], g_sems.at[j])
        for j in range(3)
    ]
    gathers.append(None)
    x_copy.wait()

    stores = []
    for j in range(N_SUB):
        gathers[j % 3].wait()

        def sub_row(p):
            r = j * SUB + p
            for c in range(EMBED_DIM // 16):
                sl = pl.ds(c * 16, 16)
                x_v[r, sl] = x_v[r, sl] - rows_v[j % 3, p, sl]

        plsc.parallel_loop(0, SUB, 1, unroll=8)(sub_row)
        stores.append(pltpu.async_copy(
            x_v.at[pl.ds(j * SUB, SUB)],
            out_hbm.at[pl.ds(base + j * SUB, SUB)], o_sem))
        if j + 3 < N_SUB:
            gathers[j % 3] = pltpu.async_copy(
                table_hbm.at[idx_v.at[pl.ds((j + 3) * SUB, SUB)]],
                rows_v.at[j % 3], g_sems.at[j % 3])
    for s in stores:
        s.wait()


@jax.jit
def _batch_correct(x, batch_labels, batch_embed):
    mesh = plsc.VectorSubcoreMesh(core_axis_name="c", subcore_axis_name="s")
    tpad = jnp.pad(batch_embed, ((0, 0), (0, 128 - EMBED_DIM)))
    return pl.kernel(
        _sc_body,
        out_type=jax.ShapeDtypeStruct((B, EMBED_DIM), jnp.float32),
        mesh=mesh,
        scratch_types=[
            pltpu.VMEM((B_PER_W,), jnp.int32),
            pltpu.VMEM((3, SUB, 128), jnp.float32),
            pltpu.VMEM((B_PER_W, EMBED_DIM), jnp.float32),
            pltpu.SemaphoreType.DMA,
            pltpu.SemaphoreType.DMA((3,)),
            pltpu.SemaphoreType.DMA,
        ],
        compiler_params=pltpu.CompilerParams(
            use_tc_tiling_on_sc=True,
            disable_bounds_checks=True,
            disable_semaphore_checks=True,
            skip_device_barrier=True,
        ),
    )(x, batch_labels, tpad)


def kernel(x, batch_labels, batch_embed):
    return _batch_correct(x, batch_labels.astype(jnp.int32), batch_embed)


# speedup vs baseline: 2.2804x; 1.0031x over previous
import jax
import jax.numpy as jnp
from jax import lax
from jax.experimental import pallas as pl
from jax.experimental.pallas import tpu as pltpu
from jax.experimental.pallas import tpu_sc as plsc

EMBED_DIM = 64
NUM_BATCHES = 1000
B = 16384

NC = 2
NS = 16
NW = NC * NS
B_PER_W = B // NW
N_SUB = 4
SUB = B_PER_W // N_SUB


def _sc_body(x_hbm, idx_hbm, table_hbm, out_hbm,
             idx_v, rows_v, x_v, x_sem, g_sems, o_sem):
    wid = lax.axis_index("s") * NC + lax.axis_index("c")
    base = wid * B_PER_W

    x_copy = pltpu.async_copy(x_hbm.at[pl.ds(base, B_PER_W)], x_v, x_sem)
    pltpu.sync_copy(idx_hbm.at[pl.ds(base, B_PER_W)], idx_v)
    gathers = [
        pltpu.async_copy(
            table_hbm.at[idx_v.at[pl.ds(j * SUB, SUB)]],
            rows_v.at[j], g_sems.at[j])
        for j in range(3)
    ]
    gathers.append(None)
    x_copy.wait()

    stores = []
    for j in range(N_SUB):
        gathers[j % 3].wait()

        def sub_row(p):
            r = j * SUB + p
            for c in range(EMBED_DIM // 16):
                sl = pl.ds(c * 16, 16)
                x_v[r, sl] = x_v[r, sl] - rows_v[j % 3, p, sl]

        plsc.parallel_loop(0, SUB, 1, unroll=8)(sub_row)
        stores.append(pltpu.async_copy(
            x_v.at[pl.ds(j * SUB, SUB)],
            out_hbm.at[pl.ds(base + j * SUB, SUB)], o_sem))
        if j + 3 < N_SUB:
            gathers[j % 3] = pltpu.async_copy(
                table_hbm.at[idx_v.at[pl.ds((j + 3) * SUB, SUB)]],
                rows_v.at[j % 3], g_sems.at[j % 3])
    for s in stores:
        s.wait()


@jax.jit
def _batch_correct(x, batch_labels, batch_embed):
    mesh = plsc.VectorSubcoreMesh(core_axis_name="c", subcore_axis_name="s")
    tpad = jnp.pad(batch_embed, ((0, 0), (0, 128 - EMBED_DIM)))
    return pl.kernel(
        _sc_body,
        out_type=jax.ShapeDtypeStruct((B, EMBED_DIM), jnp.float32),
        mesh=mesh,
        scratch_types=[
            pltpu.VMEM((B_PER_W,), jnp.int32),
            pltpu.VMEM((3, SUB, 128), jnp.float32),
            pltpu.VMEM((B_PER_W, EMBED_DIM), jnp.float32),
            pltpu.SemaphoreType.DMA,
            pltpu.SemaphoreType.DMA((3,)),
            pltpu.SemaphoreType.DMA,
        ],
        compiler_params=pltpu.CompilerParams(
            use_tc_tiling_on_sc=True,
            needs_layout_passes=False,
            disable_bounds_checks=True,
            disable_semaphore_checks=True,
            skip_device_barrier=True,
        ),
    )(x, batch_labels, tpad)


def kernel(x, batch_labels, batch_embed):
    return _batch_correct(x, batch_labels.astype(jnp.int32), batch_embed)
